# baseline (device time: 56170 ns/iter reference)
import jax
import jax.numpy as jnp
from jax import lax
from jax.experimental import pallas as pl
from jax.experimental.pallas import tpu as pltpu

Y_SIZE = 2
Z_SIZE = 4
N_TOK_BLOCKS = Y_SIZE * Z_SIZE
V_TILE = 1024
N_ROUNDS = 4


def kernel(x, W, labels):
    T, D = x.shape
    V_shard = W.shape[1]
    tb = T // N_TOK_BLOCKS
    n_v_tiles = V_shard // V_TILE

    my_y = lax.axis_index("y")
    my_z = lax.axis_index("z")
    blk = my_y * Z_SIZE + my_z

    x_blk = lax.dynamic_slice(x, (blk * tb, 0), (tb, D))
    lab_blk = lax.dynamic_slice(labels, (blk * tb,), (tb,)).reshape(1, tb)

    def body(x_ref, w_ref, lab_ref, out_ref,
             s_acc, l_acc, acc, recv, send_sems, recv_sems):
        v = pl.program_id(0)
        mx = lax.axis_index("x")
        my = lax.axis_index("y")
        mz = lax.axis_index("z")

        @pl.when(v == 0)
        def _():
            s_acc[...] = jnp.zeros_like(s_acc)
            l_acc[...] = jnp.zeros_like(l_acc)

        logits = jnp.dot(x_ref[...], w_ref[...],
                         preferred_element_type=jnp.float32)
        s_acc[0, :] += jnp.sum(jnp.exp(logits), axis=1)
        col0 = mx * V_shard + v * V_TILE
        cols = col0 + lax.broadcasted_iota(jnp.int32, (tb, V_TILE), 1)
        hit = cols == lab_ref[0, :][:, None]
        l_acc[0, :] += jnp.sum(jnp.where(hit, logits, 0.0), axis=1)

        @pl.when(v == n_v_tiles - 1)
        def _():
            b = my * Z_SIZE + mz
            acc[...] = jnp.zeros_like(acc)
            acc[0, pl.ds(b, 1), :] = s_acc[...]
            acc[1, pl.ds(b, 1), :] = l_acc[...]

            partners = [
                (1 - mx, my, mz),
                (mx, 1 - my, mz),
                (mx, my, mz ^ 1),
                (mx, my, mz ^ 2),
            ]

            bar = pltpu.get_barrier_semaphore()
            for p in partners:
                pl.semaphore_signal(bar, inc=1, device_id=p,
                                    device_id_type=pl.DeviceIdType.MESH)
            pl.semaphore_wait(bar, len(partners))

            for r, p in enumerate(partners):
                rdma = pltpu.make_async_remote_copy(
                    src_ref=acc,
                    dst_ref=recv.at[r],
                    send_sem=send_sems.at[r],
                    recv_sem=recv_sems.at[r],
                    device_id=p,
                    device_id_type=pl.DeviceIdType.MESH,
                )
                rdma.start()
                rdma.wait()
                acc[...] += recv[r]

            out_ref[...] = jnp.log(acc[0]) - acc[1]

    out = pl.pallas_call(
        body,
        grid=(n_v_tiles,),
        in_specs=[
            pl.BlockSpec((tb, D), lambda v: (0, 0)),
            pl.BlockSpec((D, V_TILE), lambda v: (0, v)),
            pl.BlockSpec((1, tb), lambda v: (0, 0)),
        ],
        out_specs=pl.BlockSpec((N_TOK_BLOCKS, tb), lambda v: (0, 0)),
        out_shape=jax.ShapeDtypeStruct((N_TOK_BLOCKS, tb), jnp.float32),
        scratch_shapes=[
            pltpu.VMEM((1, tb), jnp.float32),
            pltpu.VMEM((1, tb), jnp.float32),
            pltpu.VMEM((2, N_TOK_BLOCKS, tb), jnp.float32),
            pltpu.VMEM((N_ROUNDS, 2, N_TOK_BLOCKS, tb), jnp.float32),
            pltpu.SemaphoreType.DMA((N_ROUNDS,)),
            pltpu.SemaphoreType.DMA((N_ROUNDS,)),
        ],
        compiler_params=pltpu.CompilerParams(
            dimension_semantics=("arbitrary",),
            collective_id=0,
        ),
    )(x_blk, W, lab_blk)
    return out.reshape(T)
